# Initial kernel scaffold; baseline (speedup 1.0000x reference)
import jax
import jax.numpy as jnp
from jax.experimental import pallas as pl


def kernel(x, edge_index, W1, b1, Wm, bm, Wl, bl):
    raise NotImplementedError("write your pallas kernel here")


# speedup vs baseline: 11.2902x; 11.2902x over previous
import functools

import jax
import jax.numpy as jnp
from jax import lax
from jax.experimental import pallas as pl
from jax.experimental.pallas import tpu as pltpu
from jax.experimental.pallas import tpu_sc as plsc

N = 10000
NP = 10240
E = 320000
IN_CH = 128
HID = 128
OUT = 64

NC = 2
NS = 16
NW = NC * NS
EPW = E // NW
CHUNK = 80
NCHUNK = EPW // CHUNK
RPT = NP // NS
ZR = 160
DEG_W = 16

_mesh = plsc.VectorSubcoreMesh(core_axis_name="c", subcore_axis_name="s")


_DEG_SCRATCH = [
    pltpu.VMEM((CHUNK, HID), jnp.float32),
    pltpu.VMEM((CHUNK,), jnp.int32),
    pltpu.VMEM_SHARED((NP, HID), jnp.float32),
]


def _deg_body(dst_hbm, zeros_hbm, out_hbm, ones_v, didx, acc):
    c = lax.axis_index("c")
    s = lax.axis_index("s")
    wid = c * NS + s

    def _inito(i, carry):
        for j in range(HID // 16):
            ones_v[i, pl.ds(j * 16, 16)] = jnp.ones((16,), jnp.float32)
        return carry

    lax.fori_loop(0, CHUNK, _inito, 0)

    @pl.when(s == 0)
    def _zero_acc():
        pltpu.sync_copy(zeros_hbm, acc)

    plsc.subcore_barrier()

    def _step(t, carry):
        eoff = (wid * NCHUNK + t) * CHUNK
        pltpu.sync_copy(dst_hbm.at[pl.ds(eoff, CHUNK)], didx)
        pltpu.sync_copy(ones_v, acc.at[didx], add=True)
        return carry

    lax.fori_loop(0, NCHUNK, _step, 0)

    plsc.subcore_barrier()

    @pl.when(s == 0)
    def _export():
        pltpu.sync_copy(acc, out_hbm.at[c])


_SPASS_SCRATCH = [
    pltpu.VMEM((CHUNK, HID), jnp.float32),
    pltpu.VMEM((CHUNK,), jnp.int32),
    pltpu.VMEM((CHUNK,), jnp.int32),
    pltpu.VMEM_SHARED((NP, HID), jnp.float32),
    pltpu.SemaphoreType.DMA,
]


def _spass_body(m_hbm, src_hbm, dst_hbm, zeros_hbm, out_hbm,
                rows_v, sidx, didx, acc, sem0):
    c = lax.axis_index("c")
    s = lax.axis_index("s")
    wid = c * NS + s

    @pl.when(s == 0)
    def _zero_acc():
        pltpu.sync_copy(zeros_hbm, acc)

    plsc.subcore_barrier()

    def _step(t, carry):
        eoff = (wid * NCHUNK + t) * CHUNK
        pltpu.sync_copy(src_hbm.at[pl.ds(eoff, CHUNK)], sidx)
        pltpu.sync_copy(dst_hbm.at[pl.ds(eoff, CHUNK)], didx)
        pltpu.async_copy(m_hbm.at[sidx], rows_v, sem0).wait()
        pltpu.sync_copy(rows_v, acc.at[didx], add=True)
        return carry

    lax.fori_loop(0, NCHUNK, _step, 0)

    plsc.subcore_barrier()

    @pl.when(s == 0)
    def _export():
        pltpu.sync_copy(acc, out_hbm.at[c])


_deg_kernel = pl.kernel(
    _deg_body,
    out_type=jax.ShapeDtypeStruct((NC, NP, HID), jnp.float32),
    mesh=_mesh, scratch_types=_DEG_SCRATCH)

_spass_kernel = pl.kernel(
    _spass_body,
    out_type=jax.ShapeDtypeStruct((NC, NP, HID), jnp.float32),
    mesh=_mesh, scratch_types=_SPASS_SCRATCH)


BR = 1000


def _dinv_col(degp):
    cnt = degp[0] + degp[1]
    return lax.rsqrt(cnt + 1.0)[:, 0:1]


def _clrelu(v):
    return jnp.clip(jnp.where(v >= 0, v, 0.01 * v), -3.0, 1.0)


def _l1_body(degp_ref, x_ref, w_ref, h_ref, m_ref):
    dinv = _dinv_col(degp_ref[...])
    h = jnp.dot(x_ref[...], w_ref[...], preferred_element_type=jnp.float32)
    h_ref[...] = h
    m_ref[...] = h * dinv


def _post1_body(degp_ref, h1_ref, p_ref, b_ref, xemp_ref, xh_ref, m2_ref):
    dinv = _dinv_col(degp_ref[...])
    g = p_ref[0] + p_ref[1]
    h1 = h1_ref[...]
    xemp = dinv * g + (dinv * dinv) * h1 + b_ref[...]
    xh = _clrelu(xemp)
    xemp_ref[...] = xemp
    xh_ref[...] = xh
    m2_ref[...] = xh * dinv


def _post2_body(degp_ref, xh_ref, p_ref, wm_ref, wl_ref, bm_ref, bl_ref,
                me_ref, le_ref, mean_ref, logstd_ref):
    dinv = _dinv_col(degp_ref[...])
    u = p_ref[0] + p_ref[1]
    aggx = dinv * u + (dinv * dinv) * xh_ref[...]
    me = jnp.dot(aggx, wm_ref[...], preferred_element_type=jnp.float32) + bm_ref[...]
    le = jnp.dot(aggx, wl_ref[...], preferred_element_type=jnp.float32) + bl_ref[...]
    me_ref[...] = me
    le_ref[...] = le
    mean_ref[...] = _clrelu(me)
    logstd_ref[...] = _clrelu(le)


_degp_spec = pl.BlockSpec((2, BR, HID), lambda i: (0, i, 0))
_row128_spec = pl.BlockSpec((BR, HID), lambda i: (i, 0))
_parts_spec = pl.BlockSpec((2, BR, HID), lambda i: (0, i, 0))

_l1_call = pl.pallas_call(
    _l1_body,
    grid=(N // BR,),
    in_specs=[
        _degp_spec,
        pl.BlockSpec((BR, IN_CH), lambda i: (i, 0)),
        pl.BlockSpec((IN_CH, HID), lambda i: (0, 0)),
    ],
    out_specs=[_row128_spec, _row128_spec],
    out_shape=[jax.ShapeDtypeStruct((N, HID), jnp.float32)] * 2,
)

_post1_call = pl.pallas_call(
    _post1_body,
    grid=(N // BR,),
    in_specs=[
        _degp_spec,
        _row128_spec,
        _parts_spec,
        pl.BlockSpec((1, HID), lambda i: (0, 0)),
    ],
    out_specs=[_row128_spec] * 3,
    out_shape=[jax.ShapeDtypeStruct((N, HID), jnp.float32)] * 3,
)

_post2_call = pl.pallas_call(
    _post2_body,
    grid=(N // BR,),
    in_specs=[
        _degp_spec,
        _row128_spec,
        _parts_spec,
        pl.BlockSpec((HID, OUT), lambda i: (0, 0)),
        pl.BlockSpec((HID, OUT), lambda i: (0, 0)),
        pl.BlockSpec((1, OUT), lambda i: (0, 0)),
        pl.BlockSpec((1, OUT), lambda i: (0, 0)),
    ],
    out_specs=[pl.BlockSpec((BR, OUT), lambda i: (i, 0))] * 4,
    out_shape=[jax.ShapeDtypeStruct((N, OUT), jnp.float32)] * 4,
)


def kernel(x, edge_index, W1, b1, Wm, bm, Wl, bl):
    src_flat = edge_index[0]
    dst_flat = edge_index[1]

    zs = jnp.zeros((NP, HID), jnp.float32)
    degp = _deg_kernel(dst_flat, zs)
    h1, m1 = _l1_call(degp, x, W1)
    p1 = _spass_kernel(m1, src_flat, dst_flat, zs)
    x_emp, xh, m2 = _post1_call(degp, h1, p1, b1.reshape(1, HID))
    p2 = _spass_kernel(m2, src_flat, dst_flat, zs)
    mean_emp, logstd_emp, mean, logstd = _post2_call(
        degp, xh, p2, Wm, Wl, bm.reshape(1, OUT), bl.reshape(1, OUT))
    return (xh, mean, logstd, x_emp, mean_emp, logstd_emp)


# speedup vs baseline: 16.0172x; 1.4187x over previous
import jax
import jax.numpy as jnp
from jax import lax
from jax.experimental import pallas as pl
from jax.experimental.pallas import tpu as pltpu
from jax.experimental.pallas import tpu_sc as plsc

N = 10000
NP = 10240
E = 320000
IN_CH = 128
HID = 128
OUT = 64

NC = 2
NS = 16
NW = NC * NS
EPW = E // NW
CHUNK = 80
NCHUNK = EPW // CHUNK
NBUF = 4
NOUT = NCHUNK // NBUF
NTAIL = NCHUNK - NOUT * NBUF

_mesh = plsc.VectorSubcoreMesh(core_axis_name="c", subcore_axis_name="s")


_DEG_SCRATCH = (
    [pltpu.VMEM((CHUNK, HID), jnp.float32)]
    + [pltpu.VMEM((CHUNK,), jnp.int32) for _ in range(NBUF)]
    + [pltpu.VMEM_SHARED((NP, HID), jnp.float32)]
)


def _deg_body(dst_hbm, zeros_hbm, out_hbm, ones_v, *rest):
    didx = rest[:NBUF]
    acc = rest[NBUF]
    c = lax.axis_index("c")
    s = lax.axis_index("s")
    wid = c * NS + s

    def _inito(i, carry):
        for j in range(HID // 16):
            ones_v[i, pl.ds(j * 16, 16)] = jnp.ones((16,), jnp.float32)
        return carry

    lax.fori_loop(0, CHUNK, _inito, 0)

    @pl.when(s == 0)
    def _zero_acc():
        pltpu.sync_copy(zeros_hbm, acc)

    plsc.subcore_barrier()

    def _outer(o, carry):
        base = (wid * NCHUNK + o * NBUF) * CHUNK
        for b in range(NBUF):
            pltpu.sync_copy(dst_hbm.at[pl.ds(base + b * CHUNK, CHUNK)], didx[b])
        for b in range(NBUF):
            pltpu.sync_copy(ones_v, acc.at[didx[b]], add=True)
        return carry

    lax.fori_loop(0, NOUT, _outer, 0)
    for b in range(NTAIL):
        eoff = (wid * NCHUNK + NOUT * NBUF + b) * CHUNK
        pltpu.sync_copy(dst_hbm.at[pl.ds(eoff, CHUNK)], didx[b])
        pltpu.sync_copy(ones_v, acc.at[didx[b]], add=True)

    plsc.subcore_barrier()

    @pl.when(s == 0)
    def _export():
        pltpu.sync_copy(acc, out_hbm.at[c])


_SPASS_SCRATCH = (
    [pltpu.VMEM((CHUNK, HID), jnp.float32) for _ in range(NBUF)]
    + [pltpu.VMEM((CHUNK,), jnp.int32) for _ in range(NBUF)]
    + [pltpu.VMEM((CHUNK,), jnp.int32) for _ in range(NBUF)]
    + [pltpu.SemaphoreType.DMA for _ in range(NBUF)]
    + [pltpu.VMEM_SHARED((NP, HID), jnp.float32)]
)


def _spass_body(m_hbm, src_hbm, dst_hbm, zeros_hbm, out_hbm, *rest):
    rows = rest[0:NBUF]
    sidx = rest[NBUF:2 * NBUF]
    didx = rest[2 * NBUF:3 * NBUF]
    sems = rest[3 * NBUF:4 * NBUF]
    acc = rest[4 * NBUF]
    c = lax.axis_index("c")
    s = lax.axis_index("s")
    wid = c * NS + s

    @pl.when(s == 0)
    def _zero_acc():
        pltpu.sync_copy(zeros_hbm, acc)

    plsc.subcore_barrier()

    def _outer(o, carry):
        base = (wid * NCHUNK + o * NBUF) * CHUNK
        for b in range(NBUF):
            pltpu.sync_copy(src_hbm.at[pl.ds(base + b * CHUNK, CHUNK)], sidx[b])
            pltpu.sync_copy(dst_hbm.at[pl.ds(base + b * CHUNK, CHUNK)], didx[b])
            pltpu.async_copy(m_hbm.at[sidx[b]], rows[b], sems[b])
        for b in range(NBUF):
            pltpu.make_async_copy(m_hbm.at[sidx[b]], rows[b], sems[b]).wait()
            pltpu.sync_copy(rows[b], acc.at[didx[b]], add=True)
        return carry

    lax.fori_loop(0, NOUT, _outer, 0)
    for b in range(NTAIL):
        eoff = (wid * NCHUNK + NOUT * NBUF + b) * CHUNK
        pltpu.sync_copy(src_hbm.at[pl.ds(eoff, CHUNK)], sidx[b])
        pltpu.sync_copy(dst_hbm.at[pl.ds(eoff, CHUNK)], didx[b])
        pltpu.async_copy(m_hbm.at[sidx[b]], rows[b], sems[b]).wait()
        pltpu.sync_copy(rows[b], acc.at[didx[b]], add=True)

    plsc.subcore_barrier()

    @pl.when(s == 0)
    def _export():
        pltpu.sync_copy(acc, out_hbm.at[c])


_deg_kernel = pl.kernel(
    _deg_body,
    out_type=jax.ShapeDtypeStruct((NC, NP, HID), jnp.float32),
    mesh=_mesh, scratch_types=_DEG_SCRATCH)

_spass_kernel = pl.kernel(
    _spass_body,
    out_type=jax.ShapeDtypeStruct((NC, NP, HID), jnp.float32),
    mesh=_mesh, scratch_types=_SPASS_SCRATCH)


BR = 1000


def _dinv_col(degp):
    cnt = degp[0] + degp[1]
    return lax.rsqrt(cnt + 1.0)[:, 0:1]


def _clrelu(v):
    return jnp.clip(jnp.where(v >= 0, v, 0.01 * v), -3.0, 1.0)


def _l1_body(degp_ref, x_ref, w_ref, h_ref, m_ref):
    dinv = _dinv_col(degp_ref[...])
    h = jnp.dot(x_ref[...], w_ref[...], preferred_element_type=jnp.float32)
    h_ref[...] = h
    m_ref[...] = h * dinv


def _post1_body(degp_ref, h1_ref, p_ref, b_ref, xemp_ref, xh_ref, m2_ref):
    dinv = _dinv_col(degp_ref[...])
    g = p_ref[0] + p_ref[1]
    h1 = h1_ref[...]
    xemp = dinv * g + (dinv * dinv) * h1 + b_ref[...]
    xh = _clrelu(xemp)
    xemp_ref[...] = xemp
    xh_ref[...] = xh
    m2_ref[...] = xh * dinv


def _post2_body(degp_ref, xh_ref, p_ref, wm_ref, wl_ref, bm_ref, bl_ref,
                me_ref, le_ref, mean_ref, logstd_ref):
    dinv = _dinv_col(degp_ref[...])
    u = p_ref[0] + p_ref[1]
    aggx = dinv * u + (dinv * dinv) * xh_ref[...]
    me = jnp.dot(aggx, wm_ref[...], preferred_element_type=jnp.float32) + bm_ref[...]
    le = jnp.dot(aggx, wl_ref[...], preferred_element_type=jnp.float32) + bl_ref[...]
    me_ref[...] = me
    le_ref[...] = le
    mean_ref[...] = _clrelu(me)
    logstd_ref[...] = _clrelu(le)


_degp_spec = pl.BlockSpec((2, BR, HID), lambda i: (0, i, 0))
_row128_spec = pl.BlockSpec((BR, HID), lambda i: (i, 0))
_parts_spec = pl.BlockSpec((2, BR, HID), lambda i: (0, i, 0))

_l1_call = pl.pallas_call(
    _l1_body,
    grid=(N // BR,),
    in_specs=[
        _degp_spec,
        pl.BlockSpec((BR, IN_CH), lambda i: (i, 0)),
        pl.BlockSpec((IN_CH, HID), lambda i: (0, 0)),
    ],
    out_specs=[_row128_spec, _row128_spec],
    out_shape=[jax.ShapeDtypeStruct((N, HID), jnp.float32)] * 2,
)

_post1_call = pl.pallas_call(
    _post1_body,
    grid=(N // BR,),
    in_specs=[
        _degp_spec,
        _row128_spec,
        _parts_spec,
        pl.BlockSpec((1, HID), lambda i: (0, 0)),
    ],
    out_specs=[_row128_spec] * 3,
    out_shape=[jax.ShapeDtypeStruct((N, HID), jnp.float32)] * 3,
)

_post2_call = pl.pallas_call(
    _post2_body,
    grid=(N // BR,),
    in_specs=[
        _degp_spec,
        _row128_spec,
        _parts_spec,
        pl.BlockSpec((HID, OUT), lambda i: (0, 0)),
        pl.BlockSpec((HID, OUT), lambda i: (0, 0)),
        pl.BlockSpec((1, OUT), lambda i: (0, 0)),
        pl.BlockSpec((1, OUT), lambda i: (0, 0)),
    ],
    out_specs=[pl.BlockSpec((BR, OUT), lambda i: (i, 0))] * 4,
    out_shape=[jax.ShapeDtypeStruct((N, OUT), jnp.float32)] * 4,
)


def kernel(x, edge_index, W1, b1, Wm, bm, Wl, bl):
    src_flat = edge_index[0]
    dst_flat = edge_index[1]

    zs = jnp.zeros((NP, HID), jnp.float32)
    degp = _deg_kernel(dst_flat, zs)
    h1, m1 = _l1_call(degp, x, W1)
    p1 = _spass_kernel(m1, src_flat, dst_flat, zs)
    x_emp, xh, m2 = _post1_call(degp, h1, p1, b1.reshape(1, HID))
    p2 = _spass_kernel(m2, src_flat, dst_flat, zs)
    mean_emp, logstd_emp, mean, logstd = _post2_call(
        degp, xh, p2, Wm, Wl, bm.reshape(1, OUT), bl.reshape(1, OUT))
    return (xh, mean, logstd, x_emp, mean_emp, logstd_emp)


# speedup vs baseline: 16.0719x; 1.0034x over previous
import jax
import jax.numpy as jnp
from jax import lax
from jax.experimental import pallas as pl
from jax.experimental.pallas import tpu as pltpu
from jax.experimental.pallas import tpu_sc as plsc

N = 10000
NP = 10240
E = 320000
IN_CH = 128
HID = 128
OUT = 64

NC = 2
NS = 16
NW = NC * NS
EPW = E // NW
CHUNK = 80
NCHUNK = EPW // CHUNK
NBUF = 4
NOUT = NCHUNK // NBUF
NTAIL = NCHUNK - NOUT * NBUF

_mesh = plsc.VectorSubcoreMesh(core_axis_name="c", subcore_axis_name="s")


_DEG_SCRATCH = (
    [pltpu.VMEM((CHUNK, HID), jnp.float32)]
    + [pltpu.VMEM((CHUNK,), jnp.int32) for _ in range(NBUF)]
    + [pltpu.VMEM_SHARED((NP, HID), jnp.float32)]
)


def _deg_body(dst_hbm, zeros_hbm, out_hbm, ones_v, *rest):
    didx = rest[:NBUF]
    acc = rest[NBUF]
    c = lax.axis_index("c")
    s = lax.axis_index("s")
    wid = c * NS + s

    def _inito(i, carry):
        for j in range(HID // 16):
            ones_v[i, pl.ds(j * 16, 16)] = jnp.ones((16,), jnp.float32)
        return carry

    lax.fori_loop(0, CHUNK, _inito, 0)

    @pl.when(s == 0)
    def _zero_acc():
        pltpu.sync_copy(zeros_hbm, acc)

    plsc.subcore_barrier()

    ebase = wid * NCHUNK * CHUNK

    for b in range(NBUF):
        pltpu.sync_copy(dst_hbm.at[pl.ds(ebase + b * CHUNK, CHUNK)], didx[b])

    def _outer(o, carry):
        for b in range(NBUF):
            pltpu.sync_copy(ones_v, acc.at[didx[b]], add=True)
            pltpu.sync_copy(
                dst_hbm.at[pl.ds(ebase + ((o + 1) * NBUF + b) * CHUNK, CHUNK)],
                didx[b])
        return carry

    lax.fori_loop(0, NOUT - 1, _outer, 0)
    for b in range(NBUF):
        pltpu.sync_copy(ones_v, acc.at[didx[b]], add=True)
    for b in range(NTAIL):
        pltpu.sync_copy(
            dst_hbm.at[pl.ds(ebase + (NOUT * NBUF + b) * CHUNK, CHUNK)], didx[b])
        pltpu.sync_copy(ones_v, acc.at[didx[b]], add=True)

    plsc.subcore_barrier()

    @pl.when(s == 0)
    def _export():
        pltpu.sync_copy(acc, out_hbm.at[c])


_SPASS_SCRATCH = (
    [pltpu.VMEM((CHUNK, HID), jnp.float32) for _ in range(NBUF)]
    + [pltpu.VMEM((CHUNK,), jnp.int32) for _ in range(NBUF)]
    + [pltpu.VMEM((CHUNK,), jnp.int32) for _ in range(NBUF)]
    + [pltpu.SemaphoreType.DMA for _ in range(NBUF)]
    + [pltpu.VMEM_SHARED((NP, HID), jnp.float32)]
)


def _spass_body(m_hbm, src_hbm, dst_hbm, zeros_hbm, out_hbm, *rest):
    rows = rest[0:NBUF]
    sidx = rest[NBUF:2 * NBUF]
    didx = rest[2 * NBUF:3 * NBUF]
    sems = rest[3 * NBUF:4 * NBUF]
    acc = rest[4 * NBUF]
    c = lax.axis_index("c")
    s = lax.axis_index("s")
    wid = c * NS + s

    @pl.when(s == 0)
    def _zero_acc():
        pltpu.sync_copy(zeros_hbm, acc)

    plsc.subcore_barrier()

    ebase = wid * NCHUNK * CHUNK

    def _fire(t, b):
        pltpu.sync_copy(src_hbm.at[pl.ds(ebase + t * CHUNK, CHUNK)], sidx[b])
        pltpu.sync_copy(dst_hbm.at[pl.ds(ebase + t * CHUNK, CHUNK)], didx[b])
        pltpu.async_copy(m_hbm.at[sidx[b]], rows[b], sems[b])

    def _drain(b):
        pltpu.make_async_copy(m_hbm.at[sidx[b]], rows[b], sems[b]).wait()
        pltpu.sync_copy(rows[b], acc.at[didx[b]], add=True)

    for b in range(NBUF):
        _fire(b, b)

    def _outer(o, carry):
        for b in range(NBUF):
            _drain(b)
            _fire((o + 1) * NBUF + b, b)
        return carry

    lax.fori_loop(0, NOUT - 1, _outer, 0)
    for b in range(NBUF):
        _drain(b)
    for b in range(NTAIL):
        _fire(NOUT * NBUF + b, b)
        _drain(b)

    plsc.subcore_barrier()

    @pl.when(s == 0)
    def _export():
        pltpu.sync_copy(acc, out_hbm.at[c])


_deg_kernel = pl.kernel(
    _deg_body,
    out_type=jax.ShapeDtypeStruct((NC, NP, HID), jnp.float32),
    mesh=_mesh, scratch_types=_DEG_SCRATCH)

_spass_kernel = pl.kernel(
    _spass_body,
    out_type=jax.ShapeDtypeStruct((NC, NP, HID), jnp.float32),
    mesh=_mesh, scratch_types=_SPASS_SCRATCH)


BR = 1000


def _dinv_col(degp):
    cnt = degp[0] + degp[1]
    return lax.rsqrt(cnt + 1.0)[:, 0:1]


def _clrelu(v):
    return jnp.clip(jnp.where(v >= 0, v, 0.01 * v), -3.0, 1.0)


def _l1_body(degp_ref, x_ref, w_ref, h_ref, m_ref):
    dinv = _dinv_col(degp_ref[...])
    h = jnp.dot(x_ref[...], w_ref[...], preferred_element_type=jnp.float32)
    h_ref[...] = h
    m_ref[...] = h * dinv


def _post1_body(degp_ref, h1_ref, p_ref, b_ref, xemp_ref, xh_ref, m2_ref):
    dinv = _dinv_col(degp_ref[...])
    g = p_ref[0] + p_ref[1]
    h1 = h1_ref[...]
    xemp = dinv * g + (dinv * dinv) * h1 + b_ref[...]
    xh = _clrelu(xemp)
    xemp_ref[...] = xemp
    xh_ref[...] = xh
    m2_ref[...] = xh * dinv


def _post2_body(degp_ref, xh_ref, p_ref, wm_ref, wl_ref, bm_ref, bl_ref,
                me_ref, le_ref, mean_ref, logstd_ref):
    dinv = _dinv_col(degp_ref[...])
    u = p_ref[0] + p_ref[1]
    aggx = dinv * u + (dinv * dinv) * xh_ref[...]
    me = jnp.dot(aggx, wm_ref[...], preferred_element_type=jnp.float32) + bm_ref[...]
    le = jnp.dot(aggx, wl_ref[...], preferred_element_type=jnp.float32) + bl_ref[...]
    me_ref[...] = me
    le_ref[...] = le
    mean_ref[...] = _clrelu(me)
    logstd_ref[...] = _clrelu(le)


_degp_spec = pl.BlockSpec((2, BR, HID), lambda i: (0, i, 0))
_row128_spec = pl.BlockSpec((BR, HID), lambda i: (i, 0))
_parts_spec = pl.BlockSpec((2, BR, HID), lambda i: (0, i, 0))

_l1_call = pl.pallas_call(
    _l1_body,
    grid=(N // BR,),
    in_specs=[
        _degp_spec,
        pl.BlockSpec((BR, IN_CH), lambda i: (i, 0)),
        pl.BlockSpec((IN_CH, HID), lambda i: (0, 0)),
    ],
    out_specs=[_row128_spec, _row128_spec],
    out_shape=[jax.ShapeDtypeStruct((N, HID), jnp.float32)] * 2,
)

_post1_call = pl.pallas_call(
    _post1_body,
    grid=(N // BR,),
    in_specs=[
        _degp_spec,
        _row128_spec,
        _parts_spec,
        pl.BlockSpec((1, HID), lambda i: (0, 0)),
    ],
    out_specs=[_row128_spec] * 3,
    out_shape=[jax.ShapeDtypeStruct((N, HID), jnp.float32)] * 3,
)

_post2_call = pl.pallas_call(
    _post2_body,
    grid=(N // BR,),
    in_specs=[
        _degp_spec,
        _row128_spec,
        _parts_spec,
        pl.BlockSpec((HID, OUT), lambda i: (0, 0)),
        pl.BlockSpec((HID, OUT), lambda i: (0, 0)),
        pl.BlockSpec((1, OUT), lambda i: (0, 0)),
        pl.BlockSpec((1, OUT), lambda i: (0, 0)),
    ],
    out_specs=[pl.BlockSpec((BR, OUT), lambda i: (i, 0))] * 4,
    out_shape=[jax.ShapeDtypeStruct((N, OUT), jnp.float32)] * 4,
)


def kernel(x, edge_index, W1, b1, Wm, bm, Wl, bl):
    src_flat = edge_index[0]
    dst_flat = edge_index[1]

    zs = jnp.zeros((NP, HID), jnp.float32)
    degp = _deg_kernel(dst_flat, zs)
    h1, m1 = _l1_call(degp, x, W1)
    p1 = _spass_kernel(m1, src_flat, dst_flat, zs)
    x_emp, xh, m2 = _post1_call(degp, h1, p1, b1.reshape(1, HID))
    p2 = _spass_kernel(m2, src_flat, dst_flat, zs)
    mean_emp, logstd_emp, mean, logstd = _post2_call(
        degp, xh, p2, Wm, Wl, bm.reshape(1, OUT), bl.reshape(1, OUT))
    return (xh, mean, logstd, x_emp, mean_emp, logstd_emp)


# speedup vs baseline: 24.3938x; 1.5178x over previous
import jax
import jax.numpy as jnp
from jax import lax
from jax.experimental import pallas as pl
from jax.experimental.pallas import tpu as pltpu
from jax.experimental.pallas import tpu_sc as plsc

N = 10000
NP = 10240
E = 320000
IN_CH = 128
HID = 128
OUT = 64

NC = 2
NS = 16
NW = NC * NS
EPW = E // NW
CHUNK = 80
NCHUNK = EPW // CHUNK
NBUF = 4
NOUT = NCHUNK // NBUF
NTAIL = NCHUNK - NOUT * NBUF
NGRP = NOUT

_mesh = plsc.VectorSubcoreMesh(core_axis_name="c", subcore_axis_name="s")


_DEG_SCRATCH = (
    [pltpu.VMEM((CHUNK, HID), jnp.float32)]
    + [pltpu.VMEM((CHUNK,), jnp.int32) for _ in range(2 * NBUF)]
    + [pltpu.SemaphoreType.DMA, pltpu.SemaphoreType.DMA]
    + [pltpu.VMEM_SHARED((NP, HID), jnp.float32)]
)


def _deg_body(dst_hbm, zeros_hbm, out_hbm, ones_v, *rest):
    didx = (rest[0:NBUF], rest[NBUF:2 * NBUF])
    isem = rest[2 * NBUF:2 * NBUF + 2]
    acc = rest[2 * NBUF + 2]
    c = lax.axis_index("c")
    s = lax.axis_index("s")
    wid = c * NS + s
    ebase = wid * NCHUNK * CHUNK

    def _inito(i, carry):
        for j in range(HID // 16):
            ones_v[i, pl.ds(j * 16, 16)] = jnp.ones((16,), jnp.float32)
        return carry

    lax.fori_loop(0, CHUNK, _inito, 0)

    @pl.when(s == 0)
    def _zero_acc():
        pltpu.sync_copy(zeros_hbm, acc)

    plsc.subcore_barrier()

    def _loadidx(g, S):
        base = ebase + g * NBUF * CHUNK
        for b in range(NBUF):
            pltpu.async_copy(dst_hbm.at[pl.ds(base + b * CHUNK, CHUNK)],
                             didx[S][b], isem[S])

    def _waitidx(S):
        for b in range(NBUF):
            pltpu.make_async_copy(dst_hbm.at[pl.ds(0, CHUNK)],
                                  didx[S][b], isem[S]).wait()

    def _group(g, P):
        Q = 1 - P
        for b in range(NBUF):
            pltpu.sync_copy(ones_v, acc.at[didx[P][b]], add=True)
        _loadidx(jnp.minimum(g + 2, NGRP - 1), P)
        _waitidx(Q)

    _loadidx(0, 0)
    _waitidx(0)
    _loadidx(1, 1)

    def _pair(i, carry):
        _group(2 * i, 0)
        _group(2 * i + 1, 1)
        return carry

    lax.fori_loop(0, (NGRP - 1) // 2, _pair, 0)
    _waitidx(1)
    for b in range(NBUF):
        pltpu.sync_copy(ones_v, acc.at[didx[0][b]], add=True)
    for b in range(NTAIL):
        pltpu.sync_copy(
            dst_hbm.at[pl.ds(ebase + (NGRP * NBUF + b) * CHUNK, CHUNK)],
            didx[0][b])
        pltpu.sync_copy(ones_v, acc.at[didx[0][b]], add=True)

    plsc.subcore_barrier()

    @pl.when(s == 0)
    def _export():
        pltpu.sync_copy(acc, out_hbm.at[c])


_SPASS_SCRATCH = (
    [pltpu.VMEM((CHUNK, HID), jnp.float32) for _ in range(NBUF)]
    + [pltpu.VMEM((CHUNK,), jnp.int32) for _ in range(2 * NBUF)]
    + [pltpu.VMEM((CHUNK,), jnp.int32) for _ in range(2 * NBUF)]
    + [pltpu.SemaphoreType.DMA for _ in range(NBUF)]
    + [pltpu.SemaphoreType.DMA, pltpu.SemaphoreType.DMA]
    + [pltpu.VMEM_SHARED((NP, HID), jnp.float32)]
)


def _spass_body(m_hbm, src_hbm, dst_hbm, zeros_hbm, out_hbm, *rest):
    rows = rest[0:NBUF]
    sidx = (rest[NBUF:2 * NBUF], rest[2 * NBUF:3 * NBUF])
    didx = (rest[3 * NBUF:4 * NBUF], rest[4 * NBUF:5 * NBUF])
    sems = rest[5 * NBUF:6 * NBUF]
    isem = rest[6 * NBUF:6 * NBUF + 2]
    acc = rest[6 * NBUF + 2]
    c = lax.axis_index("c")
    s = lax.axis_index("s")
    wid = c * NS + s
    ebase = wid * NCHUNK * CHUNK

    @pl.when(s == 0)
    def _zero_acc():
        pltpu.sync_copy(zeros_hbm, acc)

    plsc.subcore_barrier()

    def _loadidx(g, S):
        base = ebase + g * NBUF * CHUNK
        for b in range(NBUF):
            pltpu.async_copy(src_hbm.at[pl.ds(base + b * CHUNK, CHUNK)],
                             sidx[S][b], isem[S])
            pltpu.async_copy(dst_hbm.at[pl.ds(base + b * CHUNK, CHUNK)],
                             didx[S][b], isem[S])

    def _waitidx(S):
        for b in range(NBUF):
            pltpu.make_async_copy(src_hbm.at[pl.ds(0, CHUNK)],
                                  sidx[S][b], isem[S]).wait()
            pltpu.make_async_copy(dst_hbm.at[pl.ds(0, CHUNK)],
                                  didx[S][b], isem[S]).wait()

    def _fire(b, S):
        pltpu.async_copy(m_hbm.at[sidx[S][b]], rows[b], sems[b])

    def _drain(b, S):
        pltpu.make_async_copy(m_hbm.at[sidx[S][b]], rows[b], sems[b]).wait()
        pltpu.sync_copy(rows[b], acc.at[didx[S][b]], add=True)

    def _group(g, P):
        Q = 1 - P
        _waitidx(Q)
        for b in range(NBUF):
            _drain(b, P)
            _fire(b, Q)
        _loadidx(jnp.minimum(g + 2, NGRP - 1), P)
        return

    _loadidx(0, 0)
    _waitidx(0)
    for b in range(NBUF):
        _fire(b, 0)
    _loadidx(1, 1)

    def _pair(i, carry):
        _group(2 * i, 0)
        _group(2 * i + 1, 1)
        return carry

    lax.fori_loop(0, (NGRP - 1) // 2, _pair, 0)
    _waitidx(1)
    for b in range(NBUF):
        _drain(b, 0)
    for b in range(NTAIL):
        eoff = ebase + (NGRP * NBUF + b) * CHUNK
        pltpu.sync_copy(src_hbm.at[pl.ds(eoff, CHUNK)], sidx[0][b])
        pltpu.sync_copy(dst_hbm.at[pl.ds(eoff, CHUNK)], didx[0][b])
        pltpu.async_copy(m_hbm.at[sidx[0][b]], rows[b], sems[b]).wait()
        pltpu.sync_copy(rows[b], acc.at[didx[0][b]], add=True)

    plsc.subcore_barrier()

    @pl.when(s == 0)
    def _export():
        pltpu.sync_copy(acc, out_hbm.at[c])


_deg_kernel = pl.kernel(
    _deg_body,
    out_type=jax.ShapeDtypeStruct((NC, NP, HID), jnp.float32),
    mesh=_mesh, scratch_types=_DEG_SCRATCH)

_spass_kernel = pl.kernel(
    _spass_body,
    out_type=jax.ShapeDtypeStruct((NC, NP, HID), jnp.float32),
    mesh=_mesh, scratch_types=_SPASS_SCRATCH)


BR = 1000


def _dinv_col(degp):
    cnt = degp[0] + degp[1]
    return lax.rsqrt(cnt + 1.0)[:, 0:1]


def _clrelu(v):
    return jnp.clip(jnp.where(v >= 0, v, 0.01 * v), -3.0, 1.0)


def _l1_body(degp_ref, x_ref, w_ref, h_ref, m_ref):
    dinv = _dinv_col(degp_ref[...])
    h = jnp.dot(x_ref[...], w_ref[...], preferred_element_type=jnp.float32)
    h_ref[...] = h
    m_ref[...] = h * dinv


def _post1_body(degp_ref, h1_ref, p_ref, b_ref, xemp_ref, xh_ref, m2_ref):
    dinv = _dinv_col(degp_ref[...])
    g = p_ref[0] + p_ref[1]
    h1 = h1_ref[...]
    xemp = dinv * g + (dinv * dinv) * h1 + b_ref[...]
    xh = _clrelu(xemp)
    xemp_ref[...] = xemp
    xh_ref[...] = xh
    m2_ref[...] = xh * dinv


def _post2_body(degp_ref, xh_ref, p_ref, wm_ref, wl_ref, bm_ref, bl_ref,
                me_ref, le_ref, mean_ref, logstd_ref):
    dinv = _dinv_col(degp_ref[...])
    u = p_ref[0] + p_ref[1]
    aggx = dinv * u + (dinv * dinv) * xh_ref[...]
    me = jnp.dot(aggx, wm_ref[...], preferred_element_type=jnp.float32) + bm_ref[...]
    le = jnp.dot(aggx, wl_ref[...], preferred_element_type=jnp.float32) + bl_ref[...]
    me_ref[...] = me
    le_ref[...] = le
    mean_ref[...] = _clrelu(me)
    logstd_ref[...] = _clrelu(le)


_degp_spec = pl.BlockSpec((2, BR, HID), lambda i: (0, i, 0))
_row128_spec = pl.BlockSpec((BR, HID), lambda i: (i, 0))
_parts_spec = pl.BlockSpec((2, BR, HID), lambda i: (0, i, 0))

_l1_call = pl.pallas_call(
    _l1_body,
    grid=(N // BR,),
    in_specs=[
        _degp_spec,
        pl.BlockSpec((BR, IN_CH), lambda i: (i, 0)),
        pl.BlockSpec((IN_CH, HID), lambda i: (0, 0)),
    ],
    out_specs=[_row128_spec, _row128_spec],
    out_shape=[jax.ShapeDtypeStruct((N, HID), jnp.float32)] * 2,
)

_post1_call = pl.pallas_call(
    _post1_body,
    grid=(N // BR,),
    in_specs=[
        _degp_spec,
        _row128_spec,
        _parts_spec,
        pl.BlockSpec((1, HID), lambda i: (0, 0)),
    ],
    out_specs=[_row128_spec] * 3,
    out_shape=[jax.ShapeDtypeStruct((N, HID), jnp.float32)] * 3,
)

_post2_call = pl.pallas_call(
    _post2_body,
    grid=(N // BR,),
    in_specs=[
        _degp_spec,
        _row128_spec,
        _parts_spec,
        pl.BlockSpec((HID, OUT), lambda i: (0, 0)),
        pl.BlockSpec((HID, OUT), lambda i: (0, 0)),
        pl.BlockSpec((1, OUT), lambda i: (0, 0)),
        pl.BlockSpec((1, OUT), lambda i: (0, 0)),
    ],
    out_specs=[pl.BlockSpec((BR, OUT), lambda i: (i, 0))] * 4,
    out_shape=[jax.ShapeDtypeStruct((N, OUT), jnp.float32)] * 4,
)


def kernel(x, edge_index, W1, b1, Wm, bm, Wl, bl):
    src_flat = edge_index[0]
    dst_flat = edge_index[1]

    zs = jnp.zeros((NP, HID), jnp.float32)
    degp = _deg_kernel(dst_flat, zs)
    h1, m1 = _l1_call(degp, x, W1)
    p1 = _spass_kernel(m1, src_flat, dst_flat, zs)
    x_emp, xh, m2 = _post1_call(degp, h1, p1, b1.reshape(1, HID))
    p2 = _spass_kernel(m2, src_flat, dst_flat, zs)
    mean_emp, logstd_emp, mean, logstd = _post2_call(
        degp, xh, p2, Wm, Wl, bm.reshape(1, OUT), bl.reshape(1, OUT))
    return (xh, mean, logstd, x_emp, mean_emp, logstd_emp)
